# Initial kernel scaffold; baseline (speedup 1.0000x reference)
import jax
import jax.numpy as jnp
from jax.experimental import pallas as pl


def kernel(input):
    raise NotImplementedError("write your pallas kernel here")


# speedup vs baseline: 14.1522x; 14.1522x over previous
import functools

import jax
import jax.numpy as jnp
from jax.experimental import pallas as pl

TOPK_K = 256
ROWS = 128
COLS = 32768
BLOCK_ROWS = 8


def _topk_mask_body(x_ref, o_ref):
    x = x_ref[...]
    u = pltpu_bitcast_uint32(x)
    sign = u >> jnp.uint32(31)
    ukey = jnp.where(sign == jnp.uint32(1), ~u, u | jnp.uint32(0x80000000))

    kf = jnp.float32(TOPK_K)
    t = jnp.zeros((x.shape[0], 1), dtype=jnp.uint32)
    for b in range(31, -1, -1):
        cand = t | jnp.uint32(1 << b)
        cnt = jnp.sum(
            jnp.where(ukey >= cand, jnp.float32(1.0), jnp.float32(0.0)),
            axis=1,
            keepdims=True,
        )
        t = jnp.where(cnt >= kf, cand, t)

    ge = ukey >= t
    cnt_ge = jnp.sum(
        jnp.where(ge, jnp.float32(1.0), jnp.float32(0.0)), axis=1, keepdims=True
    )
    no_tie = jnp.all(cnt_ge == kf)

    @pl.when(no_tie)
    def _():
        o_ref[...] = jnp.where(ge, x, jnp.float32(0.0))

    @pl.when(jnp.logical_not(no_tie))
    def _():
        gt = ukey > t
        cnt_gt = jnp.sum(
            jnp.where(gt, jnp.float32(1.0), jnp.float32(0.0)),
            axis=1,
            keepdims=True,
        )
        need_eq = kf - cnt_gt
        eq = ukey == t
        idx = jax.lax.broadcasted_iota(jnp.int32, x.shape, 1)
        m = jnp.zeros((x.shape[0], 1), dtype=jnp.int32)
        for b in range(14, -1, -1):
            cand = m | jnp.int32(1 << b)
            cnt = jnp.sum(
                jnp.where(eq & (idx < cand), jnp.float32(1.0), jnp.float32(0.0)),
                axis=1,
                keepdims=True,
            )
            m = jnp.where(cnt < need_eq, cand, m)
        keep = gt | (eq & (idx <= m))
        o_ref[...] = jnp.where(keep, x, jnp.float32(0.0))


def pltpu_bitcast_uint32(x):
    return jax.lax.bitcast_convert_type(x, jnp.uint32)


@functools.partial(jax.jit)
def kernel(input):
    return pl.pallas_call(
        _topk_mask_body,
        grid=(ROWS // BLOCK_ROWS,),
        in_specs=[pl.BlockSpec((BLOCK_ROWS, COLS), lambda i: (i, 0))],
        out_specs=pl.BlockSpec((BLOCK_ROWS, COLS), lambda i: (i, 0)),
        out_shape=jax.ShapeDtypeStruct((ROWS, COLS), jnp.float32),
    )(input)
